# baseline (speedup 1.0000x reference)
import functools

import jax
import jax.numpy as jnp
from jax import lax
from jax.experimental import pallas as pl
from jax.experimental.pallas import tpu as pltpu
from jax.experimental.pallas import tpu_sc as plsc

BATCH = 128
FEATS = 32768
NC = 2
NS = 16
LANES = 16
NW = NC * NS
ROWS_PER_W = BATCH // NW
UNROLL = 4
VECS = FEATS // LANES
STEPS = VECS // UNROLL

_mesh = plsc.VectorSubcoreMesh(core_axis_name="c", subcore_axis_name="s")

_INF = float("inf")


@functools.partial(
    pl.kernel,
    mesh=_mesh,
    compiler_params=pltpu.CompilerParams(needs_layout_passes=False),
    out_type=jax.ShapeDtypeStruct((NW, LANES), jnp.float32),
    scratch_types=[
        pltpu.VMEM((FEATS,), jnp.float32),
        pltpu.VMEM((FEATS,), jnp.float32),
        pltpu.VMEM((LANES,), jnp.float32),
        pltpu.VMEM((LANES,), jnp.float32),
        pltpu.SemaphoreType.DMA,
        pltpu.SemaphoreType.DMA,
    ],
)
def _closest_sc(inp_hbm, prev_hbm, out_hbm, buf_a, buf_b, prev_v, out_v,
                sem_a, sem_b):
    c = lax.axis_index("c")
    s = lax.axis_index("s")
    wid = s * NC + c
    base_row = wid * ROWS_PER_W

    pltpu.sync_copy(prev_hbm.at[wid], prev_v)
    pvec = prev_v[...]

    bufs = (buf_a, buf_b)
    sems = (sem_a, sem_b)
    cur = pltpu.async_copy(inp_hbm.at[base_row], bufs[0], sems[0])

    lane_iota = lax.iota(jnp.int32, LANES)
    res = jnp.zeros((LANES,), jnp.float32)

    for r in range(ROWS_PER_W):
        nxt = None
        if r + 1 < ROWS_PER_W:
            nxt = pltpu.async_copy(
                inp_hbm.at[base_row + r + 1], bufs[(r + 1) % 2],
                sems[(r + 1) % 2])
        cur.wait()
        buf = bufs[r % 2]
        p = pvec[r]

        def step(i, carry, buf=buf, p=p):
            new = []
            start = i * (UNROLL * LANES)
            for k in range(UNROLL):
                bd, bv = carry[2 * k], carry[2 * k + 1]
                x = buf[pl.ds(start + k * LANES, LANES)]
                d = jnp.abs(x - p)
                better = d < bd
                new.append(jnp.where(better, d, bd))
                new.append(jnp.where(better, x, bv))
            return tuple(new)

        init = []
        for _ in range(UNROLL):
            init.append(jnp.full((LANES,), _INF, jnp.float32))
            init.append(jnp.zeros((LANES,), jnp.float32))
        carry = lax.fori_loop(0, STEPS, step, tuple(init))

        bd, bv = carry[0], carry[1]
        for k in range(1, UNROLL):
            d2, v2 = carry[2 * k], carry[2 * k + 1]
            better = d2 < bd
            bd = jnp.where(better, d2, bd)
            bv = jnp.where(better, v2, bv)
        _, vs = plsc.sort_key_val(bd, bv)
        val = vs[0]
        res = jnp.where(lane_iota == r, val, res)
        cur = nxt

    out_v[...] = res
    pltpu.sync_copy(out_v, out_hbm.at[wid])


def kernel(input, prev_output):
    prev_staged = jnp.zeros((NW, LANES), jnp.float32)
    prev_staged = prev_staged.at[:, :ROWS_PER_W].set(
        prev_output.reshape(NW, ROWS_PER_W))
    out = _closest_sc(input, prev_staged)
    return out[:, :ROWS_PER_W].reshape(BATCH, 1)
